# baseline (speedup 1.0000x reference)
import functools

import jax
import jax.numpy as jnp
from jax import lax
from jax.experimental import pallas as pl
from jax.experimental.pallas import tpu as pltpu
from jax.experimental.pallas import tpu_sc as plsc

VOCAB = 100000
EMBED_DIM = 64
BATCH = 4096
HIST = 50
N = BATCH * HIST

NUM_CORES = 2
NUM_SUBCORES = 16
NUM_WORKERS = NUM_CORES * NUM_SUBCORES
PER_WORKER = N // NUM_WORKERS
CHUNK = 640
NUM_CHUNKS = PER_WORKER // CHUNK
UNROLL = 16
WORDS = EMBED_DIM // 4

_mesh = plsc.VectorSubcoreMesh(
    core_axis_name="c", subcore_axis_name="s",
    num_cores=NUM_CORES, num_subcores=NUM_SUBCORES)

_scratch = []
for _ in range(2):
    _scratch += [
        pltpu.VMEM((CHUNK,), jnp.int32),
        pltpu.VMEM((CHUNK, WORDS), jnp.int32),
        pltpu.VMEM((CHUNK,), jnp.float32),
        pltpu.VMEM((CHUNK * EMBED_DIM,), jnp.float32),
        pltpu.SemaphoreType.DMA,
        pltpu.SemaphoreType.DMA,
        pltpu.SemaphoreType.DMA,
    ]


@functools.partial(
    pl.kernel,
    out_type=jax.ShapeDtypeStruct((N * EMBED_DIM,), jnp.float32),
    mesh=_mesh,
    scratch_types=_scratch,
    compiler_params=pltpu.CompilerParams(use_tc_tiling_on_sc=False),
)
def _sc_embed(w_hbm, s_hbm, idx_hbm, out_hbm, *bufs):
    wid = lax.axis_index("s") * NUM_CORES + lax.axis_index("c")
    base_w = wid * PER_WORKER
    B = [bufs[0:7], bufs[7:14]]

    def issue(c):
        idx_v, rows_v, scale_v, _, sem_w, sem_s, _ = B[c % 2]
        base = base_w + c * CHUNK
        pltpu.sync_copy(idx_hbm.at[pl.ds(base, CHUNK)], idx_v)
        cp_w = pltpu.async_copy(w_hbm.at[idx_v], rows_v, sem_w)
        cp_s = pltpu.async_copy(s_hbm.at[idx_v], scale_v, sem_s)
        return cp_w, cp_s

    gathers = {0: issue(0)}
    out_cps = {}
    for c in range(NUM_CHUNKS):
        idx_v, rows_v, scale_v, out_v, sem_w, sem_s, sem_o = B[c % 2]
        if c + 1 < NUM_CHUNKS:
            gathers[c + 1] = issue(c + 1)
        cp_w, cp_s = gathers.pop(c)
        cp_w.wait()
        cp_s.wait()
        if c >= 2:
            out_cps.pop(c - 2).wait()

        def row_body(i, _, rows_v=rows_v, scale_v=scale_v, out_v=out_v):
            sblk = scale_v[pl.ds(i * UNROLL, UNROLL)]
            for u in range(UNROLL):
                r = i * UNROLL + u
                w32 = rows_v[r]
                sv = lax.broadcast(sblk[u], (16,))
                for k in range(4):
                    x = (w32 << (24 - 8 * k)) >> 24
                    y = x.astype(jnp.float32) * sv
                    out_v[pl.ds(r * EMBED_DIM + k * 16, 16)] = y
            return ()

        lax.fori_loop(0, CHUNK // UNROLL, row_body, (), unroll=False)
        base = base_w + c * CHUNK
        out_cps[c] = pltpu.async_copy(
            out_v, out_hbm.at[pl.ds(base * EMBED_DIM, CHUNK * EMBED_DIM)],
            sem_o)
    for cp in out_cps.values():
        cp.wait()


def kernel(input, weight_int8, scale):
    idx = input.reshape(-1).astype(jnp.int32)
    a = input_rows = weight_int8.reshape(VOCAB, 4, 16)
    s0 = a[:, 0, :].astype(jnp.int32) & 255
    s1 = (a[:, 1, :].astype(jnp.int32) & 255) << 8
    s2 = (a[:, 2, :].astype(jnp.int32) & 255) << 16
    s3 = a[:, 3, :].astype(jnp.int32) << 24
    w32 = s0 | s1 | s2 | s3
    out = _sc_embed(w32, scale.reshape(-1), idx)
    return out.reshape(BATCH, HIST, EMBED_DIM)


# speedup vs baseline: 1.4176x; 1.4176x over previous
import functools

import jax
import jax.numpy as jnp
from jax import lax
from jax.experimental import pallas as pl
from jax.experimental.pallas import tpu as pltpu
from jax.experimental.pallas import tpu_sc as plsc

VOCAB = 100000
EMBED_DIM = 64
BATCH = 4096
HIST = 50
N = BATCH * HIST

NUM_CORES = 2
NUM_SUBCORES = 16
NUM_WORKERS = NUM_CORES * NUM_SUBCORES
PER_WORKER = N // NUM_WORKERS
CHUNK = 640
NUM_CHUNKS = PER_WORKER // CHUNK
UNROLL = 16
WORDS = EMBED_DIM // 4

_mesh = plsc.VectorSubcoreMesh(
    core_axis_name="c", subcore_axis_name="s",
    num_cores=NUM_CORES, num_subcores=NUM_SUBCORES)

_scratch = []
for _ in range(2):
    _scratch += [
        pltpu.VMEM((CHUNK,), jnp.int32),
        pltpu.VMEM((CHUNK, WORDS), jnp.int32),
        pltpu.VMEM((CHUNK,), jnp.float32),
        pltpu.VMEM((CHUNK * EMBED_DIM,), jnp.float32),
        pltpu.SemaphoreType.DMA,
        pltpu.SemaphoreType.DMA,
        pltpu.SemaphoreType.DMA,
    ]


@functools.partial(
    pl.kernel,
    out_type=jax.ShapeDtypeStruct((N * EMBED_DIM,), jnp.float32),
    mesh=_mesh,
    scratch_types=_scratch,
    compiler_params=pltpu.CompilerParams(use_tc_tiling_on_sc=False),
)
def _sc_embed(w_hbm, s_hbm, idx_hbm, out_hbm, *bufs):
    wid = lax.axis_index("s") * NUM_CORES + lax.axis_index("c")
    base_w = wid * PER_WORKER
    B = [bufs[0:7], bufs[7:14]]

    def issue(c):
        idx_v, rows_v, scale_v, _, sem_w, sem_s, _ = B[c % 2]
        base = base_w + c * CHUNK
        pltpu.sync_copy(idx_hbm.at[pl.ds(base, CHUNK)], idx_v)
        cp_w = pltpu.async_copy(w_hbm.at[idx_v], rows_v, sem_w)
        cp_s = pltpu.async_copy(s_hbm.at[idx_v], scale_v, sem_s)
        return cp_w, cp_s

    gathers = {0: issue(0)}
    out_cps = {}
    for c in range(NUM_CHUNKS):
        idx_v, rows_v, scale_v, out_v, sem_w, sem_s, sem_o = B[c % 2]
        if c + 1 < NUM_CHUNKS:
            gathers[c + 1] = issue(c + 1)
        cp_w, cp_s = gathers.pop(c)
        cp_w.wait()
        cp_s.wait()
        if c >= 2:
            out_cps.pop(c - 2).wait()

        def row_body(i, _, rows_v=rows_v, scale_v=scale_v, out_v=out_v):
            sblk = scale_v[pl.ds(i * UNROLL, UNROLL)]
            for u in range(UNROLL):
                r = i * UNROLL + u
                w32 = rows_v[r]
                sv = lax.broadcast(sblk[u], (16,))
                for k in range(4):
                    x = (w32 << (24 - 8 * k)) >> 24
                    y = x.astype(jnp.float32) * sv
                    out_v[pl.ds(r * EMBED_DIM + k * 16, 16)] = y
            return ()

        lax.fori_loop(0, CHUNK // UNROLL, row_body, (), unroll=False)
        base = base_w + c * CHUNK
        out_cps[c] = pltpu.async_copy(
            out_v, out_hbm.at[pl.ds(base * EMBED_DIM, CHUNK * EMBED_DIM)],
            sem_o)
    for cp in out_cps.values():
        cp.wait()


def kernel(input, weight_int8, scale):
    idx = input.reshape(-1).astype(jnp.int32)
    wp = weight_int8.reshape(VOCAB, 4, 16).transpose(0, 2, 1)
    w32 = lax.bitcast_convert_type(wp, jnp.int32)
    out = _sc_embed(w32, scale.reshape(-1), idx)
    return out.reshape(BATCH, HIST, EMBED_DIM)
